# Initial kernel scaffold; baseline (speedup 1.0000x reference)
import jax
import jax.numpy as jnp
from jax.experimental import pallas as pl


def kernel(per_atom_energy, atomic_subsystem_indices):
    raise NotImplementedError("write your pallas kernel here")


# speedup vs baseline: 15.8808x; 15.8808x over previous
import functools

import jax
import jax.numpy as jnp
from jax import lax
from jax.experimental import pallas as pl
from jax.experimental.pallas import tpu as pltpu
from jax.experimental.pallas import tpu_sc as plsc

N_ATOMS = 3_200_000
N_MOL = 50_000
STD = 1.2
MEAN = -0.5

LANES = 16
NC = 2
NS = 16
NW = NC * NS
PER_W = N_ATOMS // NW
CHUNK = 10_000
NCHUNK = PER_W // CHUNK
VECS = CHUNK // LANES

ACC_N = ((N_MOL + 255) // 256) * 256
SLICE = ACC_N // NS
SLICE_V = SLICE // LANES


def _sc_body(e_hbm, i_hbm, part_hbm, acc, ebuf, ibuf, mbuf, rbuf, shared):
    c = lax.axis_index("c")
    s = lax.axis_index("s")
    wid = s * NC + c
    base = wid * PER_W

    zero = jnp.zeros((LANES,), jnp.float32)

    def zbody(i, _):
        acc[pl.ds(i * LANES, LANES)] = zero
        return 0

    lax.fori_loop(0, ACC_N // LANES, zbody, 0)

    def do_chunk(ch, _):
        off = base + ch * CHUNK
        pltpu.sync_copy(e_hbm.at[pl.ds(off, CHUNK)], ebuf)
        pltpu.sync_copy(i_hbm.at[pl.ds(off, CHUNK)], ibuf)

        def vbody(i, _):
            e = ebuf[pl.ds(i * LANES, LANES)]
            ix = ibuf[pl.ds(i * LANES, LANES)]
            v = e * STD + MEAN
            plsc.addupdate_scatter(acc, [ix], v)
            return 0

        lax.fori_loop(0, VECS, vbody, 0)
        return 0

    lax.fori_loop(0, NCHUNK, do_chunk, 0)

    pltpu.sync_copy(acc, shared.at[pl.ds(s * ACC_N, ACC_N)])
    plsc.subcore_barrier()

    moff = s * SLICE
    pltpu.sync_copy(shared.at[pl.ds(moff, SLICE)], rbuf)
    for t in range(1, NS):
        pltpu.sync_copy(shared.at[pl.ds(t * ACC_N + moff, SLICE)], mbuf)

        def abody(i, _):
            o = i * LANES
            rbuf[pl.ds(o, LANES)] = rbuf[pl.ds(o, LANES)] + mbuf[pl.ds(o, LANES)]
            return 0

        lax.fori_loop(0, SLICE_V, abody, 0)

    pltpu.sync_copy(rbuf, part_hbm.at[pl.ds(c * ACC_N + moff, SLICE)])


_sc_call = functools.partial(
    pl.kernel,
    out_type=jax.ShapeDtypeStruct((NC * ACC_N,), jnp.float32),
    mesh=plsc.VectorSubcoreMesh(
        core_axis_name="c", subcore_axis_name="s", num_cores=NC, num_subcores=NS
    ),
    scratch_types=[
        pltpu.VMEM((ACC_N,), jnp.float32),
        pltpu.VMEM((CHUNK,), jnp.float32),
        pltpu.VMEM((CHUNK,), jnp.int32),
        pltpu.VMEM((SLICE,), jnp.float32),
        pltpu.VMEM((SLICE,), jnp.float32),
        pltpu.MemorySpace.VMEM_SHARED((NS * ACC_N,), jnp.float32),
    ],
    compiler_params=pltpu.CompilerParams(needs_layout_passes=False),
)(_sc_body)


def _merge_body(p_ref, o_ref):
    o_ref[...] = p_ref[0, :N_MOL] + p_ref[1, :N_MOL]


_merge_call = pl.pallas_call(
    _merge_body,
    out_shape=jax.ShapeDtypeStruct((N_MOL,), jnp.float32),
)


def kernel(per_atom_energy, atomic_subsystem_indices):
    e = per_atom_energy.reshape(N_ATOMS)
    idx = atomic_subsystem_indices.astype(jnp.int32)
    partials = _sc_call(e, idx).reshape(NC, ACC_N)
    per_molecule = _merge_call(partials)
    return (per_molecule, jax.lax.stop_gradient(per_atom_energy))


# speedup vs baseline: 18.8008x; 1.1839x over previous
import functools

import jax
import jax.numpy as jnp
from jax import lax
from jax.experimental import pallas as pl
from jax.experimental.pallas import tpu as pltpu
from jax.experimental.pallas import tpu_sc as plsc

N_ATOMS = 3_200_000
N_MOL = 50_000
STD = 1.2
MEAN = -0.5

LANES = 16
NC = 2
NS = 16
NW = NC * NS
PER_W = N_ATOMS // NW
CHUNK = 4_000
NCHUNK = PER_W // CHUNK
VECS = CHUNK // LANES
UNROLL = 25

ACC_N = ((N_MOL + 255) // 256) * 256
SLICE = ACC_N // NS
SLICE_V = SLICE // LANES


def _sc_body(e_hbm, i_hbm, part_hbm, acc, eb0, ib0, eb1, ib1,
             mb0, mb1, rbuf, shared, se0, si0, se1, si1, sm0, sm1):
    c = lax.axis_index("c")
    s = lax.axis_index("s")
    wid = s * NC + c
    base = wid * PER_W

    bufs = ((eb0, ib0, se0, si0), (eb1, ib1, se1, si1))

    def start(ch):
        eb, ib, se, si = bufs[ch % 2]
        off = base + ch * CHUNK
        d0 = pltpu.async_copy(e_hbm.at[pl.ds(off, CHUNK)], eb, se)
        d1 = pltpu.async_copy(i_hbm.at[pl.ds(off, CHUNK)], ib, si)
        return (d0, d1)

    pend = [start(0), start(1)]

    zero = jnp.zeros((LANES,), jnp.float32)

    @plsc.parallel_loop(0, ACC_N // LANES, step=1, unroll=16)
    def _(i):
        acc[pl.ds(i * LANES, LANES)] = zero

    for ch in range(NCHUNK):
        d0, d1 = pend[ch]
        d0.wait()
        d1.wait()
        eb, ib, _, _ = bufs[ch % 2]

        def vbody(k, _):
            for j in range(UNROLL):
                o = (k * UNROLL + j) * LANES
                e = eb[pl.ds(o, LANES)]
                ix = ib[pl.ds(o, LANES)]
                plsc.addupdate_scatter(acc, [ix], e * STD + MEAN)
            return 0

        lax.fori_loop(0, VECS // UNROLL, vbody, 0)
        if ch + 2 < NCHUNK:
            pend.append(start(ch + 2))

    pltpu.sync_copy(acc, shared.at[pl.ds(s * ACC_N, ACC_N)])
    plsc.subcore_barrier()

    moff = s * SLICE
    pltpu.sync_copy(shared.at[pl.ds(moff, SLICE)], rbuf)
    mbufs = ((mb0, sm0), (mb1, sm1))

    def mstart(t):
        mb, sm = mbufs[t % 2]
        return pltpu.async_copy(shared.at[pl.ds(t * ACC_N + moff, SLICE)], mb, sm)

    mpend = [None, mstart(1)]
    for t in range(1, NS):
        mpend[t % 2].wait()
        mb, _ = mbufs[t % 2]
        if t + 1 < NS:
            mpend[(t + 1) % 2] = mstart(t + 1)

        @plsc.parallel_loop(0, SLICE_V, step=1, unroll=14)
        def _(i):
            o = i * LANES
            rbuf[pl.ds(o, LANES)] = rbuf[pl.ds(o, LANES)] + mb[pl.ds(o, LANES)]

    pltpu.sync_copy(rbuf, part_hbm.at[pl.ds(c * ACC_N + moff, SLICE)])


_sc_call = functools.partial(
    pl.kernel,
    out_type=jax.ShapeDtypeStruct((NC * ACC_N,), jnp.float32),
    mesh=plsc.VectorSubcoreMesh(
        core_axis_name="c", subcore_axis_name="s", num_cores=NC, num_subcores=NS
    ),
    scratch_types=[
        pltpu.VMEM((ACC_N,), jnp.float32),
        pltpu.VMEM((CHUNK,), jnp.float32),
        pltpu.VMEM((CHUNK,), jnp.int32),
        pltpu.VMEM((CHUNK,), jnp.float32),
        pltpu.VMEM((CHUNK,), jnp.int32),
        pltpu.VMEM((SLICE,), jnp.float32),
        pltpu.VMEM((SLICE,), jnp.float32),
        pltpu.VMEM((SLICE,), jnp.float32),
        pltpu.MemorySpace.VMEM_SHARED((NS * ACC_N,), jnp.float32),
        pltpu.SemaphoreType.DMA,
        pltpu.SemaphoreType.DMA,
        pltpu.SemaphoreType.DMA,
        pltpu.SemaphoreType.DMA,
        pltpu.SemaphoreType.DMA,
        pltpu.SemaphoreType.DMA,
    ],
    compiler_params=pltpu.CompilerParams(needs_layout_passes=False),
)(_sc_body)


def _merge_body(p_ref, o_ref):
    o_ref[...] = p_ref[0, :N_MOL] + p_ref[1, :N_MOL]


_merge_call = pl.pallas_call(
    _merge_body,
    out_shape=jax.ShapeDtypeStruct((N_MOL,), jnp.float32),
)


def kernel(per_atom_energy, atomic_subsystem_indices):
    e = per_atom_energy.reshape(N_ATOMS)
    idx = atomic_subsystem_indices.astype(jnp.int32)
    partials = _sc_call(e, idx).reshape(NC, ACC_N)
    per_molecule = _merge_call(partials)
    return (per_molecule, jax.lax.stop_gradient(per_atom_energy))


# speedup vs baseline: 34.5683x; 1.8387x over previous
import functools

import jax
import jax.numpy as jnp
from jax import lax
from jax.experimental import pallas as pl
from jax.experimental.pallas import tpu as pltpu
from jax.experimental.pallas import tpu_sc as plsc

N_ATOMS = 3_200_000
N_MOL = 50_000
STD = 1.2
MEAN = -0.5

LANES = 16
NC = 2
NS = 16
NW = NC * NS
PER_W = N_ATOMS // NW
CHUNK = 2_000
NCHUNK = PER_W // CHUNK
STRIDE = CHUNK // LANES
UNROLL = 5

ACC_N = ((N_MOL + 255) // 256) * 256
SLICE = ACC_N // NS
SLICE_V = SLICE // LANES


def _sc_body(e_hbm, i_hbm, part_hbm, acc, eb0, ib0, eb1, ib1,
             mb0, mb1, rbuf, shared, se0, si0, se1, si1, sm0, sm1):
    c = lax.axis_index("c")
    s = lax.axis_index("s")
    wid = s * NC + c
    base = wid * PER_W

    bufs = ((eb0, ib0, se0, si0), (eb1, ib1, se1, si1))

    def start(ch):
        eb, ib, se, si = bufs[ch % 2]
        off = base + ch * CHUNK
        d0 = pltpu.async_copy(e_hbm.at[pl.ds(off, CHUNK)], eb, se)
        d1 = pltpu.async_copy(i_hbm.at[pl.ds(off, CHUNK)], ib, si)
        return (d0, d1)

    pend = [start(0), start(1)]

    zero = jnp.zeros((LANES,), jnp.float32)

    @plsc.parallel_loop(0, ACC_N // LANES, step=1, unroll=16)
    def _(i):
        acc[pl.ds(i * LANES, LANES)] = zero

    base_ix = lax.iota(jnp.int32, LANES) * STRIDE
    for ch in range(NCHUNK):
        d0, d1 = pend[ch]
        d0.wait()
        d1.wait()
        eb, ib, _, _ = bufs[ch % 2]

        cur_ix0 = plsc.load_gather(ib, [base_ix])

        def vbody(k, carry):
            cur_ix, cur_acc = carry
            for j in range(UNROLL):
                iv = base_ix + (k * UNROLL + j)
                e = plsc.load_gather(eb, [iv])
                ix = plsc.load_gather(ib, [iv])
                v = e * STD + MEAN
                diff = ix != cur_ix
                plsc.addupdate_scatter(acc, [cur_ix], cur_acc, mask=diff)
                cur_acc = jnp.where(diff, v, cur_acc + v)
                cur_ix = ix
            return (cur_ix, cur_acc)

        cur_ix, cur_acc = lax.fori_loop(
            0, STRIDE // UNROLL, vbody, (cur_ix0, jnp.zeros((LANES,), jnp.float32))
        )
        plsc.addupdate_scatter(acc, [cur_ix], cur_acc)
        if ch + 2 < NCHUNK:
            pend.append(start(ch + 2))

    pltpu.sync_copy(acc, shared.at[pl.ds(s * ACC_N, ACC_N)])
    plsc.subcore_barrier()

    moff = s * SLICE
    pltpu.sync_copy(shared.at[pl.ds(moff, SLICE)], rbuf)
    mbufs = ((mb0, sm0), (mb1, sm1))

    def mstart(t):
        mb, sm = mbufs[t % 2]
        return pltpu.async_copy(shared.at[pl.ds(t * ACC_N + moff, SLICE)], mb, sm)

    mpend = [None, mstart(1)]
    for t in range(1, NS):
        mpend[t % 2].wait()
        mb, _ = mbufs[t % 2]
        if t + 1 < NS:
            mpend[(t + 1) % 2] = mstart(t + 1)

        @plsc.parallel_loop(0, SLICE_V, step=1, unroll=14)
        def _(i):
            o = i * LANES
            rbuf[pl.ds(o, LANES)] = rbuf[pl.ds(o, LANES)] + mb[pl.ds(o, LANES)]

    pltpu.sync_copy(rbuf, part_hbm.at[pl.ds(c * ACC_N + moff, SLICE)])


_sc_call = functools.partial(
    pl.kernel,
    out_type=jax.ShapeDtypeStruct((NC * ACC_N,), jnp.float32),
    mesh=plsc.VectorSubcoreMesh(
        core_axis_name="c", subcore_axis_name="s", num_cores=NC, num_subcores=NS
    ),
    scratch_types=[
        pltpu.VMEM((ACC_N,), jnp.float32),
        pltpu.VMEM((CHUNK,), jnp.float32),
        pltpu.VMEM((CHUNK,), jnp.int32),
        pltpu.VMEM((CHUNK,), jnp.float32),
        pltpu.VMEM((CHUNK,), jnp.int32),
        pltpu.VMEM((SLICE,), jnp.float32),
        pltpu.VMEM((SLICE,), jnp.float32),
        pltpu.VMEM((SLICE,), jnp.float32),
        pltpu.MemorySpace.VMEM_SHARED((NS * ACC_N,), jnp.float32),
        pltpu.SemaphoreType.DMA,
        pltpu.SemaphoreType.DMA,
        pltpu.SemaphoreType.DMA,
        pltpu.SemaphoreType.DMA,
        pltpu.SemaphoreType.DMA,
        pltpu.SemaphoreType.DMA,
    ],
    compiler_params=pltpu.CompilerParams(needs_layout_passes=False),
)(_sc_body)


def _merge_body(p_ref, o_ref):
    o_ref[...] = p_ref[0, :N_MOL] + p_ref[1, :N_MOL]


_merge_call = pl.pallas_call(
    _merge_body,
    out_shape=jax.ShapeDtypeStruct((N_MOL,), jnp.float32),
)


def kernel(per_atom_energy, atomic_subsystem_indices):
    e = per_atom_energy.reshape(N_ATOMS)
    idx = atomic_subsystem_indices.astype(jnp.int32)
    partials = _sc_call(e, idx).reshape(NC, ACC_N)
    per_molecule = _merge_call(partials)
    return (per_molecule, jax.lax.stop_gradient(per_atom_energy))


# speedup vs baseline: 49.3770x; 1.4284x over previous
import functools

import jax
import jax.numpy as jnp
from jax import lax
from jax.experimental import pallas as pl
from jax.experimental.pallas import tpu as pltpu
from jax.experimental.pallas import tpu_sc as plsc

N_ATOMS = 3_200_000
N_MOL = 50_000
STD = 1.2
MEAN = -0.5

LANES = 16
NC = 2
NS = 16
NW = NC * NS
PER_W = N_ATOMS // NW
CHUNK = 2_000
NCHUNK = PER_W // CHUNK
STRIDE = CHUNK // LANES
UNROLL = 25

ACC_N = ((N_MOL + 255) // 256) * 256
SLICE = ACC_N // NS
SLICE_V = SLICE // LANES


def _sc_body(e_hbm, i_hbm, part_hbm, acc, eb0, ib0, eb1, ib1,
             mb0, mb1, rbuf, shared, se0, si0, se1, si1, sm0, sm1):
    c = lax.axis_index("c")
    s = lax.axis_index("s")
    wid = s * NC + c
    base = wid * PER_W

    bufs = ((eb0, ib0, se0, si0), (eb1, ib1, se1, si1))

    def start(ch, which):
        eb, ib, se, si = bufs[which]
        off = base + ch * CHUNK
        pltpu.async_copy(e_hbm.at[pl.ds(off, CHUNK)], eb, se)
        pltpu.async_copy(i_hbm.at[pl.ds(off, CHUNK)], ib, si)

    def wait(which):
        eb, ib, se, si = bufs[which]
        pltpu.make_async_copy(e_hbm.at[pl.ds(0, CHUNK)], eb, se).wait()
        pltpu.make_async_copy(i_hbm.at[pl.ds(0, CHUNK)], ib, si).wait()

    start(0, 0)
    start(1, 1)

    zero = jnp.zeros((LANES,), jnp.float32)

    @plsc.parallel_loop(0, ACC_N // LANES, step=1, unroll=16)
    def _(i):
        acc[pl.ds(i * LANES, LANES)] = zero

    base_ix = lax.iota(jnp.int32, LANES) * STRIDE

    def compute(which):
        eb, ib, _, _ = bufs[which]
        cur_ix0 = plsc.load_gather(ib, [base_ix])

        @plsc.parallel_loop(
            0, STRIDE, step=1, unroll=UNROLL,
            carry=(cur_ix0, jnp.zeros((LANES,), jnp.float32)),
        )
        def fin(i, carry):
            cur_ix, cur_acc = carry
            iv = base_ix + i
            e = plsc.load_gather(eb, [iv])
            ix = plsc.load_gather(ib, [iv])
            v = e * STD + MEAN
            diff = ix != cur_ix
            plsc.addupdate_scatter(acc, [cur_ix], cur_acc, mask=diff)
            return (ix, jnp.where(diff, v, cur_acc + v))

        cur_ix, cur_acc = fin
        plsc.addupdate_scatter(acc, [cur_ix], cur_acc)

    def pair(p, _):
        ch0 = p * 2
        wait(0)
        compute(0)

        @pl.when(ch0 + 2 < NCHUNK)
        def _():
            start(ch0 + 2, 0)

        wait(1)
        compute(1)

        @pl.when(ch0 + 3 < NCHUNK)
        def _():
            start(ch0 + 3, 1)

        return 0

    lax.fori_loop(0, NCHUNK // 2, pair, 0)

    pltpu.sync_copy(acc, shared.at[pl.ds(s * ACC_N, ACC_N)])
    plsc.subcore_barrier()

    moff = s * SLICE
    pltpu.sync_copy(shared.at[pl.ds(moff, SLICE)], rbuf)
    mbufs = ((mb0, sm0), (mb1, sm1))

    def mstart(t):
        mb, sm = mbufs[t % 2]
        return pltpu.async_copy(shared.at[pl.ds(t * ACC_N + moff, SLICE)], mb, sm)

    mpend = [None, mstart(1)]
    for t in range(1, NS):
        mpend[t % 2].wait()
        mb, _ = mbufs[t % 2]
        if t + 1 < NS:
            mpend[(t + 1) % 2] = mstart(t + 1)

        @plsc.parallel_loop(0, SLICE_V, step=1, unroll=14)
        def _(i):
            o = i * LANES
            rbuf[pl.ds(o, LANES)] = rbuf[pl.ds(o, LANES)] + mb[pl.ds(o, LANES)]

    pltpu.sync_copy(rbuf, part_hbm.at[pl.ds(c * ACC_N + moff, SLICE)])


_sc_call = functools.partial(
    pl.kernel,
    out_type=jax.ShapeDtypeStruct((NC * ACC_N,), jnp.float32),
    mesh=plsc.VectorSubcoreMesh(
        core_axis_name="c", subcore_axis_name="s", num_cores=NC, num_subcores=NS
    ),
    scratch_types=[
        pltpu.VMEM((ACC_N,), jnp.float32),
        pltpu.VMEM((CHUNK,), jnp.float32),
        pltpu.VMEM((CHUNK,), jnp.int32),
        pltpu.VMEM((CHUNK,), jnp.float32),
        pltpu.VMEM((CHUNK,), jnp.int32),
        pltpu.VMEM((SLICE,), jnp.float32),
        pltpu.VMEM((SLICE,), jnp.float32),
        pltpu.VMEM((SLICE,), jnp.float32),
        pltpu.MemorySpace.VMEM_SHARED((NS * ACC_N,), jnp.float32),
        pltpu.SemaphoreType.DMA,
        pltpu.SemaphoreType.DMA,
        pltpu.SemaphoreType.DMA,
        pltpu.SemaphoreType.DMA,
        pltpu.SemaphoreType.DMA,
        pltpu.SemaphoreType.DMA,
    ],
    compiler_params=pltpu.CompilerParams(needs_layout_passes=False),
)(_sc_body)


def _merge_body(p_ref, o_ref):
    o_ref[...] = p_ref[0, :N_MOL] + p_ref[1, :N_MOL]


_merge_call = pl.pallas_call(
    _merge_body,
    out_shape=jax.ShapeDtypeStruct((N_MOL,), jnp.float32),
)


def kernel(per_atom_energy, atomic_subsystem_indices):
    e = per_atom_energy.reshape(N_ATOMS)
    idx = atomic_subsystem_indices.astype(jnp.int32)
    partials = _sc_call(e, idx).reshape(NC, ACC_N)
    per_molecule = _merge_call(partials)
    return (per_molecule, jax.lax.stop_gradient(per_atom_energy))
